# speedup vs baseline: 1.9429x; 1.9429x over previous
import jax
import jax.numpy as jnp
from jax.experimental import pallas as pl
from jax.experimental.pallas import tpu as pltpu

_N = 8192
_BLOCK_ROWS = 256


def _scan_kernel(x_ref, o_ref):
    y = x_ref[...]
    lane = jax.lax.broadcasted_iota(jnp.int32, y.shape, dimension=1)
    k = 1
    while k < _N:
        shifted = pltpu.roll(y, k, axis=1)
        y = y + jnp.where(lane >= k, shifted, 0.0)
        k *= 2
    o_ref[...] = y


def kernel(x):
    m, n = x.shape
    grid = (m // _BLOCK_ROWS,)
    return pl.pallas_call(
        _scan_kernel,
        grid=grid,
        in_specs=[pl.BlockSpec((_BLOCK_ROWS, n), lambda i: (i, 0))],
        out_specs=pl.BlockSpec((_BLOCK_ROWS, n), lambda i: (i, 0)),
        out_shape=jax.ShapeDtypeStruct((m, n), x.dtype),
        compiler_params=pltpu.CompilerParams(
            dimension_semantics=("arbitrary",),
        ),
    )(x)


# speedup vs baseline: 1.9429x; 1.0000x over previous
import jax
import jax.numpy as jnp
from jax.experimental import pallas as pl
from jax.experimental.pallas import tpu as pltpu

_N = 8192
_BLOCK_ROWS = 256


def _scan_kernel(x_ref, o_ref):
    y = x_ref[...]
    lane = jax.lax.broadcasted_iota(jnp.int32, y.shape, dimension=1)
    k = 1
    while k < _N:
        shifted = pltpu.roll(y, k, axis=1)
        y = y + jnp.where(lane >= k, shifted, 0.0)
        k *= 2
    o_ref[...] = y


def kernel(x):
    m, n = x.shape
    grid = (m // _BLOCK_ROWS,)
    return pl.pallas_call(
        _scan_kernel,
        grid=grid,
        in_specs=[pl.BlockSpec((_BLOCK_ROWS, n), lambda i: (i, 0))],
        out_specs=pl.BlockSpec((_BLOCK_ROWS, n), lambda i: (i, 0)),
        out_shape=jax.ShapeDtypeStruct((m, n), x.dtype),
        compiler_params=pltpu.CompilerParams(
            dimension_semantics=("parallel",),
        ),
    )(x)


# speedup vs baseline: 5.1836x; 2.6680x over previous
import jax
import jax.numpy as jnp
from jax.experimental import pallas as pl
from jax.experimental.pallas import tpu as pltpu

_N = 8192
_CHUNK = 256
_NCHUNK = _N // _CHUNK
_BLOCK_ROWS = 128


def _split(v):
    hi = v.astype(jnp.bfloat16)
    lo = (v - hi.astype(jnp.float32)).astype(jnp.bfloat16)
    return hi, lo


def _dot2(a, b_bf16):
    ah, al = _split(a)
    return (
        jnp.dot(ah, b_bf16, preferred_element_type=jnp.float32)
        + jnp.dot(al, b_bf16, preferred_element_type=jnp.float32)
    )


def _scan_kernel(x_ref, o_ref):
    x = x_ref[...]

    ii = jax.lax.broadcasted_iota(jnp.int32, (_CHUNK, _CHUNK), 0)
    jj = jax.lax.broadcasted_iota(jnp.int32, (_CHUNK, _CHUNK), 1)
    tri = (ii <= jj).astype(jnp.bfloat16)

    xh, xl = _split(x)
    locals_ = []
    for c in range(_NCHUNK):
        s = slice(c * _CHUNK, (c + 1) * _CHUNK)
        lc = jnp.dot(xh[:, s], tri, preferred_element_type=jnp.float32)
        lc = lc + jnp.dot(xl[:, s], tri, preferred_element_type=jnp.float32)
        locals_.append(lc)
    local = jnp.concatenate(locals_, axis=1)

    totals = jnp.concatenate(
        [lc[:, _CHUNK - 1 : _CHUNK] for lc in locals_], axis=1
    )

    ci = jax.lax.broadcasted_iota(jnp.int32, (_NCHUNK, _NCHUNK), 0)
    cj = jax.lax.broadcasted_iota(jnp.int32, (_NCHUNK, _NCHUNK), 1)
    stri = (ci < cj).astype(jnp.bfloat16)
    carries = _dot2(totals, stri)

    bi = jax.lax.broadcasted_iota(jnp.int32, (_NCHUNK, _N), 0)
    bj = jax.lax.broadcasted_iota(jnp.int32, (_NCHUNK, _N), 1)
    expand = (bj // _CHUNK == bi).astype(jnp.bfloat16)
    o_ref[...] = local + _dot2(carries, expand)


def kernel(x):
    m, n = x.shape
    grid = (m // _BLOCK_ROWS,)
    return pl.pallas_call(
        _scan_kernel,
        grid=grid,
        in_specs=[pl.BlockSpec((_BLOCK_ROWS, n), lambda i: (i, 0))],
        out_specs=pl.BlockSpec((_BLOCK_ROWS, n), lambda i: (i, 0)),
        out_shape=jax.ShapeDtypeStruct((m, n), x.dtype),
        compiler_params=pltpu.CompilerParams(
            dimension_semantics=("parallel",),
        ),
    )(x)
